# baseline (speedup 1.0000x reference)
import functools

import jax
import jax.numpy as jnp
import numpy as np
from jax import lax
from jax.experimental import pallas as pl
from jax.experimental.pallas import tpu as pltpu
from jax.experimental.pallas import tpu_sc as plsc

D_MODEL = 768
N_HEADS = 12
HEAD_DIM = 64
E = 8
TOP_K = 2
D_FF = 3072
T = 2048
CAP = 512
TC = T // 2
NSLOT = E * CAP
SENT = 2 * NSLOT
NDISP = 2 * NSLOT + 512

_RB = 256


def _ln(x, g, b):
    m = jnp.mean(x, axis=-1, keepdims=True)
    v = jnp.mean((x - m) ** 2, axis=-1, keepdims=True)
    return (x - m) * jax.lax.rsqrt(v + 1e-5) * g + b


def _k_ln_qkv(x_ref, g_ref, b_ref, wq_ref, wk_ref, wv_ref,
              q_ref, k_ref, v_ref):
    h = _ln(x_ref[...], g_ref[...], b_ref[...])
    q_ref[...] = jnp.dot(h, wq_ref[...], preferred_element_type=jnp.float32)
    k_ref[...] = jnp.dot(h, wk_ref[...], preferred_element_type=jnp.float32)
    v_ref[...] = jnp.dot(h, wv_ref[...], preferred_element_type=jnp.float32)


def _k_attn(q_ref, k_ref, v_ref, o_ref):
    scale = (1.0 / np.sqrt(HEAD_DIM)).astype(np.float32)
    for p in range(2):
        sl = slice(p * HEAD_DIM, (p + 1) * HEAD_DIM)
        q = q_ref[:, sl] * scale
        k = k_ref[:, sl]
        v = v_ref[:, sl]
        s = jax.lax.dot_general(q, k, (((1,), (1,)), ((), ())),
                                preferred_element_type=jnp.float32)
        s = s - jnp.max(s, axis=-1, keepdims=True)
        e = jnp.exp(s)
        denom = jnp.sum(e, axis=-1, keepdims=True)
        o = jnp.dot(e, v, preferred_element_type=jnp.float32)
        o_ref[:, sl] = o * (1.0 / denom)


def _k_proj_ln2_gate(o_ref, wo_ref, x_ref, g_ref, b_ref, wg_ref,
                     x2_ref, mi_ref, lg_ref):
    x2 = x_ref[...] + jnp.dot(o_ref[...], wo_ref[...],
                              preferred_element_type=jnp.float32)
    x2_ref[...] = x2
    mi = _ln(x2, g_ref[...], b_ref[...])
    mi_ref[...] = mi
    lg_ref[...] = jnp.dot(mi, wg_ref[...], preferred_element_type=jnp.float32)


def _k_router(lg_ref, sa_ref, sb_ref, wa_ref, wb_ref, cnt_ref):
    lg = lg_ref[:, :E]
    m = jnp.max(lg, axis=1, keepdims=True)
    ex = jnp.exp(lg - m)
    p = ex / jnp.sum(ex, axis=1, keepdims=True)

    ie = jax.lax.broadcasted_iota(jnp.int32, (TC, E), 1)
    w1 = jnp.max(p, axis=1, keepdims=True)
    a1 = jnp.min(jnp.where(p == w1, ie, E), axis=1, keepdims=True)
    p2 = jnp.where(ie == a1, -jnp.inf, p)
    w2 = jnp.max(p2, axis=1, keepdims=True)
    a2 = jnp.min(jnp.where(p2 == w2, ie, E), axis=1, keepdims=True)
    ws = w1 + w2
    wa_ref[...] = w1 / ws
    wb_ref[...] = w2 / ws

    oha = (ie == a1).astype(jnp.float32)
    ohb = (ie == a2).astype(jnp.float32)
    cnt_ref[...] = jnp.sum(oha + ohb, axis=0).astype(jnp.int32).reshape(1, 1, E)
    ir = jax.lax.broadcasted_iota(jnp.int32, (TC, TC), 0)
    ic = jax.lax.broadcasted_iota(jnp.int32, (TC, TC), 1)
    ltri = (ir > ic).astype(jnp.float32)
    cex = jnp.dot(ltri, oha + ohb, preferred_element_type=jnp.float32)
    pos_a = jnp.sum(cex * oha, axis=1, keepdims=True)
    pos_b = jnp.sum(cex * ohb, axis=1, keepdims=True)
    cofs = pl.program_id(0) * NSLOT
    slot_a = cofs + a1 * CAP + pos_a.astype(jnp.int32)
    slot_b = cofs + a2 * CAP + pos_b.astype(jnp.int32)
    sa_ref[...] = jnp.where(pos_a < CAP, slot_a, SENT)
    sb_ref[...] = jnp.where(pos_b < CAP, slot_b, SENT)


def _sc_dispatch(mi_hbm, slots_hbm, disp_hbm, idx_v, rows_v, sem):
    wid = lax.axis_index("s") * 2 + lax.axis_index("c")
    a_base = wid * 128
    row_base = (a_base // (2 * TC)) * TC + a_base % TC
    pltpu.sync_copy(slots_hbm.at[pl.ds(a_base, 128)], idx_v)
    pltpu.sync_copy(mi_hbm.at[pl.ds(row_base, 128)], rows_v)
    pltpu.async_copy(rows_v, disp_hbm.at[idx_v], sem).wait()


FBLK = 128


def _k_ffn(cnt_ref, d_ref, w1_ref, b1_ref, w2_ref, b2_ref, o_ref):
    e = pl.program_id(0)
    c = pl.program_id(1)
    b = pl.program_id(2)
    cnt = cnt_ref[c * E + e]

    @pl.when(b * FBLK < cnt)
    def _compute():
        d = d_ref[...]
        d = jnp.where(jnp.abs(d) < 1e3, d, 0.0).astype(jnp.bfloat16)
        h = jnp.dot(d, w1_ref[0].astype(jnp.bfloat16),
                    preferred_element_type=jnp.float32)
        h = jax.nn.gelu(h + b1_ref[0])
        o_ref[...] = (jnp.dot(h.astype(jnp.bfloat16),
                              w2_ref[0].astype(jnp.bfloat16),
                              preferred_element_type=jnp.float32)
                      + b2_ref[0])

    @pl.when(b * FBLK >= cnt)
    def _skip():
        o_ref[...] = jnp.zeros_like(o_ref)


def _k_combine(sa_ref, sb_ref, wa_ref, wb_ref, eo_ref, x2_ref, out_ref):
    rb = sa_ref.shape[0]
    isl = (jax.lax.broadcasted_iota(jnp.int32, (rb, NSLOT), 1)
           + pl.program_id(0) * NSLOT)
    g = (jnp.where(sa_ref[...] == isl, wa_ref[...], 0.0)
         + jnp.where(sb_ref[...] == isl, wb_ref[...], 0.0)).astype(jnp.bfloat16)
    out_ref[...] = x2_ref[...] + jnp.dot(g, eo_ref[...].astype(jnp.bfloat16),
                                         preferred_element_type=jnp.float32)


def kernel(x, g1, bn1, g2, bn2, Wq, Wk, Wv, Wo, Wg, W1, be1, W2, be2):
    f32 = jnp.float32
    g1r, bn1r = g1.reshape(1, -1), bn1.reshape(1, -1)
    g2r, bn2r = g2.reshape(1, -1), bn2.reshape(1, -1)
    wg_pad = jnp.pad(Wg, ((0, 0), (0, 128 - E)))
    be1r = be1.reshape(E, 1, D_FF)
    be2r = be2.reshape(E, 1, D_MODEL)

    full = lambda shp: pl.BlockSpec(shp, lambda *_: tuple(0 for _ in shp))

    q, k, v = pl.pallas_call(
        _k_ln_qkv,
        grid=(T // _RB,),
        in_specs=[
            pl.BlockSpec((_RB, D_MODEL), lambda i: (i, 0)),
            full((1, D_MODEL)), full((1, D_MODEL)),
            full((D_MODEL, D_MODEL)), full((D_MODEL, D_MODEL)),
            full((D_MODEL, D_MODEL)),
        ],
        out_specs=[pl.BlockSpec((_RB, D_MODEL), lambda i: (i, 0))] * 3,
        out_shape=[jax.ShapeDtypeStruct((T, D_MODEL), f32)] * 3,
    )(x, g1r, bn1r, Wq, Wk, Wv)

    o = pl.pallas_call(
        _k_attn,
        grid=(N_HEADS // 2,),
        in_specs=[pl.BlockSpec((T, 2 * HEAD_DIM), lambda h: (0, h))] * 3,
        out_specs=pl.BlockSpec((T, 2 * HEAD_DIM), lambda h: (0, h)),
        out_shape=jax.ShapeDtypeStruct((T, D_MODEL), f32),
    )(q, k, v)

    x2, mi, logits = pl.pallas_call(
        _k_proj_ln2_gate,
        grid=(T // _RB,),
        in_specs=[
            pl.BlockSpec((_RB, D_MODEL), lambda i: (i, 0)),
            full((D_MODEL, D_MODEL)),
            pl.BlockSpec((_RB, D_MODEL), lambda i: (i, 0)),
            full((1, D_MODEL)), full((1, D_MODEL)),
            full((D_MODEL, 128)),
        ],
        out_specs=[
            pl.BlockSpec((_RB, D_MODEL), lambda i: (i, 0)),
            pl.BlockSpec((_RB, D_MODEL), lambda i: (i, 0)),
            pl.BlockSpec((_RB, 128), lambda i: (i, 0)),
        ],
        out_shape=[
            jax.ShapeDtypeStruct((T, D_MODEL), f32),
            jax.ShapeDtypeStruct((T, D_MODEL), f32),
            jax.ShapeDtypeStruct((T, 128), f32),
        ],
    )(o, Wo, x, g2r, bn2r, wg_pad)

    slot_a, slot_b, w_a, w_b, counts = pl.pallas_call(
        _k_router,
        grid=(2,),
        in_specs=[pl.BlockSpec((TC, 128), lambda c: (c, 0))],
        out_specs=[pl.BlockSpec((TC, 1), lambda c: (c, 0))] * 4
        + [pl.BlockSpec((1, 1, E), lambda c: (c, 0, 0))],
        out_shape=[
            jax.ShapeDtypeStruct((T, 1), jnp.int32),
            jax.ShapeDtypeStruct((T, 1), jnp.int32),
            jax.ShapeDtypeStruct((T, 1), f32),
            jax.ShapeDtypeStruct((T, 1), f32),
            jax.ShapeDtypeStruct((2, 1, E), jnp.int32),
        ],
    )(logits)
    counts_flat = counts.reshape(2 * E)

    sa2 = slot_a.reshape(2, TC)
    sb2 = slot_b.reshape(2, TC)
    slots = jnp.stack([sa2, sb2], axis=1).reshape(4 * TC)
    disp = pl.kernel(
        _sc_dispatch,
        mesh=plsc.VectorSubcoreMesh(core_axis_name="c", subcore_axis_name="s"),
        out_type=jax.ShapeDtypeStruct((NDISP, D_MODEL), f32),
        scratch_types=[
            pltpu.VMEM((128,), jnp.int32),
            pltpu.VMEM((128, D_MODEL), f32),
            pltpu.SemaphoreType.DMA,
        ],
    )(mi, slots)

    cb = CAP // FBLK
    eo = pl.pallas_call(
        _k_ffn,
        grid_spec=pltpu.PrefetchScalarGridSpec(
            num_scalar_prefetch=1,
            grid=(E, 2, cb),
            in_specs=[
                pl.BlockSpec((FBLK, D_MODEL),
                             lambda e, c, b, _: ((c * E + e) * cb + b, 0)),
                pl.BlockSpec((1, D_MODEL, D_FF), lambda e, c, b, _: (e, 0, 0)),
                pl.BlockSpec((1, 1, D_FF), lambda e, c, b, _: (e, 0, 0)),
                pl.BlockSpec((1, D_FF, D_MODEL), lambda e, c, b, _: (e, 0, 0)),
                pl.BlockSpec((1, 1, D_MODEL), lambda e, c, b, _: (e, 0, 0)),
            ],
            out_specs=pl.BlockSpec((FBLK, D_MODEL),
                                   lambda e, c, b, _: ((c * E + e) * cb + b, 0)),
        ),
        out_shape=jax.ShapeDtypeStruct((2 * NSLOT, D_MODEL), f32),
    )(counts_flat, disp, W1, be1r, W2, be2r)

    rb7 = 256
    out = pl.pallas_call(
        _k_combine,
        grid=(2, TC // rb7),
        in_specs=[
            pl.BlockSpec((rb7, 1), lambda c, i: (c * (TC // rb7) + i, 0)),
            pl.BlockSpec((rb7, 1), lambda c, i: (c * (TC // rb7) + i, 0)),
            pl.BlockSpec((rb7, 1), lambda c, i: (c * (TC // rb7) + i, 0)),
            pl.BlockSpec((rb7, 1), lambda c, i: (c * (TC // rb7) + i, 0)),
            pl.BlockSpec((NSLOT, D_MODEL), lambda c, i: (c, 0)),
            pl.BlockSpec((rb7, D_MODEL), lambda c, i: (c * (TC // rb7) + i, 0)),
        ],
        out_specs=pl.BlockSpec((rb7, D_MODEL), lambda c, i: (c * (TC // rb7) + i, 0)),
        out_shape=jax.ShapeDtypeStruct((T, D_MODEL), f32),
    )(slot_a, slot_b, w_a, w_b, eo, x2)

    return out


# speedup vs baseline: 1.0517x; 1.0517x over previous
import functools

import jax
import jax.numpy as jnp
import numpy as np
from jax import lax
from jax.experimental import pallas as pl
from jax.experimental.pallas import tpu as pltpu
from jax.experimental.pallas import tpu_sc as plsc

D_MODEL = 768
N_HEADS = 12
HEAD_DIM = 64
E = 8
TOP_K = 2
D_FF = 3072
T = 2048
CAP = 512
TC = T // 2
NSLOT = E * CAP
SENT = 2 * NSLOT
NDISP = 2 * NSLOT + 512

_RB = 256


def _ln(x, g, b):
    m = jnp.mean(x, axis=-1, keepdims=True)
    v = jnp.mean((x - m) ** 2, axis=-1, keepdims=True)
    return (x - m) * jax.lax.rsqrt(v + 1e-5) * g + b


def _k_ln_qkv(x_ref, g_ref, b_ref, wq_ref, wk_ref, wv_ref,
              q_ref, k_ref, v_ref):
    h = _ln(x_ref[...], g_ref[...], b_ref[...])
    q_ref[...] = jnp.dot(h, wq_ref[...], preferred_element_type=jnp.float32)
    k_ref[...] = jnp.dot(h, wk_ref[...], preferred_element_type=jnp.float32)
    v_ref[...] = jnp.dot(h, wv_ref[...], preferred_element_type=jnp.float32)


def _k_attn(q_ref, k_ref, v_ref, o_ref):
    scale = (1.0 / np.sqrt(HEAD_DIM)).astype(np.float32)
    for p in range(2):
        sl = slice(p * HEAD_DIM, (p + 1) * HEAD_DIM)
        q = q_ref[:, sl] * scale
        k = k_ref[:, sl]
        v = v_ref[:, sl]
        s = jax.lax.dot_general(q, k, (((1,), (1,)), ((), ())),
                                preferred_element_type=jnp.float32)
        s = s - jnp.max(s, axis=-1, keepdims=True)
        e = jnp.exp(s)
        denom = jnp.sum(e, axis=-1, keepdims=True)
        o = jnp.dot(e, v, preferred_element_type=jnp.float32)
        o_ref[:, sl] = o * (1.0 / denom)


def _k_proj_ln2_gate(o_ref, wo_ref, x_ref, g_ref, b_ref, wg_ref,
                     x2_ref, mi_ref, lg_ref):
    x2 = x_ref[...] + jnp.dot(o_ref[...], wo_ref[...],
                              preferred_element_type=jnp.float32)
    x2_ref[...] = x2
    mi = _ln(x2, g_ref[...], b_ref[...])
    mi_ref[...] = mi
    lg_ref[...] = jnp.dot(mi, wg_ref[...], preferred_element_type=jnp.float32)


def _k_router(lg_ref, sa_ref, sb_ref, wa_ref, wb_ref, cnt_ref):
    lg = lg_ref[:, :E]
    m = jnp.max(lg, axis=1, keepdims=True)
    ex = jnp.exp(lg - m)
    p = ex / jnp.sum(ex, axis=1, keepdims=True)

    ie = jax.lax.broadcasted_iota(jnp.int32, (TC, E), 1)
    w1 = jnp.max(p, axis=1, keepdims=True)
    a1 = jnp.min(jnp.where(p == w1, ie, E), axis=1, keepdims=True)
    p2 = jnp.where(ie == a1, -jnp.inf, p)
    w2 = jnp.max(p2, axis=1, keepdims=True)
    a2 = jnp.min(jnp.where(p2 == w2, ie, E), axis=1, keepdims=True)
    ws = w1 + w2
    wa_ref[...] = w1 / ws
    wb_ref[...] = w2 / ws

    oha = (ie == a1).astype(jnp.float32)
    ohb = (ie == a2).astype(jnp.float32)
    cnt_ref[...] = jnp.sum(oha + ohb, axis=0).astype(jnp.int32).reshape(1, 1, E)
    ir = jax.lax.broadcasted_iota(jnp.int32, (TC, TC), 0)
    ic = jax.lax.broadcasted_iota(jnp.int32, (TC, TC), 1)
    ltri = (ir > ic).astype(jnp.float32)
    cex = jnp.dot(ltri, oha + ohb, preferred_element_type=jnp.float32)
    pos_a = jnp.sum(cex * oha, axis=1, keepdims=True)
    pos_b = jnp.sum(cex * ohb, axis=1, keepdims=True)
    cofs = pl.program_id(0) * NSLOT
    slot_a = cofs + a1 * CAP + pos_a.astype(jnp.int32)
    slot_b = cofs + a2 * CAP + pos_b.astype(jnp.int32)
    sa_ref[...] = jnp.where(pos_a < CAP, slot_a, SENT)
    sb_ref[...] = jnp.where(pos_b < CAP, slot_b, SENT)


def _sc_dispatch(mi_hbm, slots_hbm, disp_hbm, idx_v, rows_v, sem):
    wid = lax.axis_index("s") * 2 + lax.axis_index("c")
    a_base = wid * 128
    row_base = (a_base // (2 * TC)) * TC + a_base % TC
    pltpu.sync_copy(slots_hbm.at[pl.ds(a_base, 128)], idx_v)
    pltpu.sync_copy(mi_hbm.at[pl.ds(row_base, 128)], rows_v)
    pltpu.async_copy(rows_v, disp_hbm.at[idx_v], sem).wait()


FBLK = 256


def _k_ffn(cnt_ref, d_ref, w1_ref, b1_ref, w2_ref, b2_ref, o_ref):
    e = pl.program_id(0)
    c = pl.program_id(1)
    b = pl.program_id(2)
    cnt = cnt_ref[c * E + e]

    @pl.when(b * FBLK < cnt)
    def _compute():
        d = d_ref[...]
        d = jnp.where(jnp.abs(d) < 1e3, d, 0.0).astype(jnp.bfloat16)
        h = jnp.dot(d, w1_ref[0].astype(jnp.bfloat16),
                    preferred_element_type=jnp.float32)
        h = jax.nn.gelu(h + b1_ref[0])
        o_ref[...] = (jnp.dot(h.astype(jnp.bfloat16),
                              w2_ref[0].astype(jnp.bfloat16),
                              preferred_element_type=jnp.float32)
                      + b2_ref[0])

    @pl.when(b * FBLK >= cnt)
    def _skip():
        o_ref[...] = jnp.zeros_like(o_ref)


def _k_combine(sa_ref, sb_ref, wa_ref, wb_ref, eo_ref, x2_ref, out_ref):
    rb = sa_ref.shape[0]
    isl = (jax.lax.broadcasted_iota(jnp.int32, (rb, NSLOT), 1)
           + pl.program_id(0) * NSLOT)
    g = (jnp.where(sa_ref[...] == isl, wa_ref[...], 0.0)
         + jnp.where(sb_ref[...] == isl, wb_ref[...], 0.0)).astype(jnp.bfloat16)
    out_ref[...] = x2_ref[...] + jnp.dot(g, eo_ref[...].astype(jnp.bfloat16),
                                         preferred_element_type=jnp.float32)


def kernel(x, g1, bn1, g2, bn2, Wq, Wk, Wv, Wo, Wg, W1, be1, W2, be2):
    f32 = jnp.float32
    g1r, bn1r = g1.reshape(1, -1), bn1.reshape(1, -1)
    g2r, bn2r = g2.reshape(1, -1), bn2.reshape(1, -1)
    wg_pad = jnp.pad(Wg, ((0, 0), (0, 128 - E)))
    be1r = be1.reshape(E, 1, D_FF)
    be2r = be2.reshape(E, 1, D_MODEL)

    full = lambda shp: pl.BlockSpec(shp, lambda *_: tuple(0 for _ in shp))

    q, k, v = pl.pallas_call(
        _k_ln_qkv,
        grid=(T // _RB,),
        in_specs=[
            pl.BlockSpec((_RB, D_MODEL), lambda i: (i, 0)),
            full((1, D_MODEL)), full((1, D_MODEL)),
            full((D_MODEL, D_MODEL)), full((D_MODEL, D_MODEL)),
            full((D_MODEL, D_MODEL)),
        ],
        out_specs=[pl.BlockSpec((_RB, D_MODEL), lambda i: (i, 0))] * 3,
        out_shape=[jax.ShapeDtypeStruct((T, D_MODEL), f32)] * 3,
    )(x, g1r, bn1r, Wq, Wk, Wv)

    o = pl.pallas_call(
        _k_attn,
        grid=(N_HEADS // 2,),
        in_specs=[pl.BlockSpec((T, 2 * HEAD_DIM), lambda h: (0, h))] * 3,
        out_specs=pl.BlockSpec((T, 2 * HEAD_DIM), lambda h: (0, h)),
        out_shape=jax.ShapeDtypeStruct((T, D_MODEL), f32),
    )(q, k, v)

    x2, mi, logits = pl.pallas_call(
        _k_proj_ln2_gate,
        grid=(T // _RB,),
        in_specs=[
            pl.BlockSpec((_RB, D_MODEL), lambda i: (i, 0)),
            full((D_MODEL, D_MODEL)),
            pl.BlockSpec((_RB, D_MODEL), lambda i: (i, 0)),
            full((1, D_MODEL)), full((1, D_MODEL)),
            full((D_MODEL, 128)),
        ],
        out_specs=[
            pl.BlockSpec((_RB, D_MODEL), lambda i: (i, 0)),
            pl.BlockSpec((_RB, D_MODEL), lambda i: (i, 0)),
            pl.BlockSpec((_RB, 128), lambda i: (i, 0)),
        ],
        out_shape=[
            jax.ShapeDtypeStruct((T, D_MODEL), f32),
            jax.ShapeDtypeStruct((T, D_MODEL), f32),
            jax.ShapeDtypeStruct((T, 128), f32),
        ],
    )(o, Wo, x, g2r, bn2r, wg_pad)

    slot_a, slot_b, w_a, w_b, counts = pl.pallas_call(
        _k_router,
        grid=(2,),
        in_specs=[pl.BlockSpec((TC, 128), lambda c: (c, 0))],
        out_specs=[pl.BlockSpec((TC, 1), lambda c: (c, 0))] * 4
        + [pl.BlockSpec((1, 1, E), lambda c: (c, 0, 0))],
        out_shape=[
            jax.ShapeDtypeStruct((T, 1), jnp.int32),
            jax.ShapeDtypeStruct((T, 1), jnp.int32),
            jax.ShapeDtypeStruct((T, 1), f32),
            jax.ShapeDtypeStruct((T, 1), f32),
            jax.ShapeDtypeStruct((2, 1, E), jnp.int32),
        ],
    )(logits)
    counts_flat = counts.reshape(2 * E)

    sa2 = slot_a.reshape(2, TC)
    sb2 = slot_b.reshape(2, TC)
    slots = jnp.stack([sa2, sb2], axis=1).reshape(4 * TC)
    disp = pl.kernel(
        _sc_dispatch,
        mesh=plsc.VectorSubcoreMesh(core_axis_name="c", subcore_axis_name="s"),
        out_type=jax.ShapeDtypeStruct((NDISP, D_MODEL), f32),
        scratch_types=[
            pltpu.VMEM((128,), jnp.int32),
            pltpu.VMEM((128, D_MODEL), f32),
            pltpu.SemaphoreType.DMA,
        ],
    )(mi, slots)

    cb = CAP // FBLK
    eo = pl.pallas_call(
        _k_ffn,
        grid_spec=pltpu.PrefetchScalarGridSpec(
            num_scalar_prefetch=1,
            grid=(E, 2, cb),
            in_specs=[
                pl.BlockSpec((FBLK, D_MODEL),
                             lambda e, c, b, _: ((c * E + e) * cb + b, 0)),
                pl.BlockSpec((1, D_MODEL, D_FF), lambda e, c, b, _: (e, 0, 0)),
                pl.BlockSpec((1, 1, D_FF), lambda e, c, b, _: (e, 0, 0)),
                pl.BlockSpec((1, D_FF, D_MODEL), lambda e, c, b, _: (e, 0, 0)),
                pl.BlockSpec((1, 1, D_MODEL), lambda e, c, b, _: (e, 0, 0)),
            ],
            out_specs=pl.BlockSpec((FBLK, D_MODEL),
                                   lambda e, c, b, _: ((c * E + e) * cb + b, 0)),
        ),
        out_shape=jax.ShapeDtypeStruct((2 * NSLOT, D_MODEL), f32),
    )(counts_flat, disp, W1, be1r, W2, be2r)

    rb7 = 256
    out = pl.pallas_call(
        _k_combine,
        grid=(2, TC // rb7),
        in_specs=[
            pl.BlockSpec((rb7, 1), lambda c, i: (c * (TC // rb7) + i, 0)),
            pl.BlockSpec((rb7, 1), lambda c, i: (c * (TC // rb7) + i, 0)),
            pl.BlockSpec((rb7, 1), lambda c, i: (c * (TC // rb7) + i, 0)),
            pl.BlockSpec((rb7, 1), lambda c, i: (c * (TC // rb7) + i, 0)),
            pl.BlockSpec((NSLOT, D_MODEL), lambda c, i: (c, 0)),
            pl.BlockSpec((rb7, D_MODEL), lambda c, i: (c * (TC // rb7) + i, 0)),
        ],
        out_specs=pl.BlockSpec((rb7, D_MODEL), lambda c, i: (c * (TC // rb7) + i, 0)),
        out_shape=jax.ShapeDtypeStruct((T, D_MODEL), f32),
    )(slot_a, slot_b, w_a, w_b, eo, x2)

    return out


# speedup vs baseline: 1.1306x; 1.0750x over previous
import functools

import jax
import jax.numpy as jnp
import numpy as np
from jax import lax
from jax.experimental import pallas as pl
from jax.experimental.pallas import tpu as pltpu
from jax.experimental.pallas import tpu_sc as plsc

D_MODEL = 768
N_HEADS = 12
HEAD_DIM = 64
E = 8
TOP_K = 2
D_FF = 3072
T = 2048
CAP = 512
TC = T // 2
NSLOT = E * CAP
SENT = 2 * NSLOT
NDISP = 2 * NSLOT + 512

_RB = 256


def _ln(x, g, b):
    m = jnp.mean(x, axis=-1, keepdims=True)
    v = jnp.mean((x - m) ** 2, axis=-1, keepdims=True)
    return (x - m) * jax.lax.rsqrt(v + 1e-5) * g + b


def _k_ln_qkv(x_ref, g_ref, b_ref, wq_ref, wk_ref, wv_ref,
              q_ref, k_ref, v_ref):
    h = _ln(x_ref[...], g_ref[...], b_ref[...])
    q_ref[...] = jnp.dot(h, wq_ref[...], preferred_element_type=jnp.float32)
    k_ref[...] = jnp.dot(h, wk_ref[...], preferred_element_type=jnp.float32)
    v_ref[...] = jnp.dot(h, wv_ref[...], preferred_element_type=jnp.float32)


def _k_attn(q_ref, k_ref, v_ref, o_ref):
    scale = (1.0 / np.sqrt(HEAD_DIM)).astype(np.float32)
    for p in range(2):
        sl = slice(p * HEAD_DIM, (p + 1) * HEAD_DIM)
        q = q_ref[:, sl] * scale
        k = k_ref[:, sl]
        v = v_ref[:, sl]
        s = jax.lax.dot_general(q, k, (((1,), (1,)), ((), ())),
                                preferred_element_type=jnp.float32)
        s = s - jnp.max(s, axis=-1, keepdims=True)
        e = jnp.exp(s)
        denom = jnp.sum(e, axis=-1, keepdims=True)
        o = jnp.dot(e, v, preferred_element_type=jnp.float32)
        o_ref[:, sl] = o * (1.0 / denom)


def _k_proj_ln2_gate(o_ref, wo_ref, x_ref, g_ref, b_ref, wg_ref,
                     x2_ref, mi_ref, lg_ref):
    x2 = x_ref[...] + jnp.dot(o_ref[...], wo_ref[...],
                              preferred_element_type=jnp.float32)
    x2_ref[...] = x2
    mi = _ln(x2, g_ref[...], b_ref[...])
    mi_ref[...] = mi
    lg_ref[...] = jnp.dot(mi, wg_ref[...], preferred_element_type=jnp.float32)


def _k_router(lg_ref, sa_ref, sb_ref, wa_ref, wb_ref, cnt_ref,
              sga_ref, sgb_ref):
    lg = lg_ref[:, :E]
    m = jnp.max(lg, axis=1, keepdims=True)
    ex = jnp.exp(lg - m)
    p = ex / jnp.sum(ex, axis=1, keepdims=True)

    ie = jax.lax.broadcasted_iota(jnp.int32, (TC, E), 1)
    w1 = jnp.max(p, axis=1, keepdims=True)
    a1 = jnp.min(jnp.where(p == w1, ie, E), axis=1, keepdims=True)
    p2 = jnp.where(ie == a1, -jnp.inf, p)
    w2 = jnp.max(p2, axis=1, keepdims=True)
    a2 = jnp.min(jnp.where(p2 == w2, ie, E), axis=1, keepdims=True)
    ws = w1 + w2

    oha = (ie == a1).astype(jnp.float32)
    ohb = (ie == a2).astype(jnp.float32)
    cnt_ref[...] = jnp.sum(oha + ohb, axis=0).astype(jnp.int32).reshape(1, 1, E)
    ir = jax.lax.broadcasted_iota(jnp.int32, (TC, TC), 0)
    ic = jax.lax.broadcasted_iota(jnp.int32, (TC, TC), 1)
    ltri = (ir > ic).astype(jnp.float32)
    cex = jnp.dot(ltri, oha + ohb, preferred_element_type=jnp.float32)
    pos_a = jnp.sum(cex * oha, axis=1, keepdims=True)
    pos_b = jnp.sum(cex * ohb, axis=1, keepdims=True)
    cofs = pl.program_id(0) * NSLOT
    keep_a = pos_a < CAP
    keep_b = pos_b < CAP
    slot_a = cofs + a1 * CAP + pos_a.astype(jnp.int32)
    slot_b = cofs + a2 * CAP + pos_b.astype(jnp.int32)
    sa_ref[...] = jnp.where(keep_a, slot_a, SENT)
    sb_ref[...] = jnp.where(keep_b, slot_b, SENT)
    sga_ref[...] = jnp.where(keep_a, slot_a, 0)
    sgb_ref[...] = jnp.where(keep_b, slot_b, 0)
    wa_ref[...] = jnp.where(keep_a, w1 / ws, 0.0)
    wb_ref[...] = jnp.where(keep_b, w2 / ws, 0.0)


def _sc_dispatch(mi_hbm, slots_hbm, disp_hbm, idx_v, rows_v, sem):
    wid = lax.axis_index("s") * 2 + lax.axis_index("c")
    a_base = wid * 128
    row_base = (a_base // (2 * TC)) * TC + a_base % TC
    pltpu.sync_copy(slots_hbm.at[pl.ds(a_base, 128)], idx_v)
    pltpu.sync_copy(mi_hbm.at[pl.ds(row_base, 128)], rows_v)
    pltpu.async_copy(rows_v, disp_hbm.at[idx_v], sem).wait()


def _k_ffn(d_ref, w1_ref, b1_ref, w2_ref, b2_ref, o_ref):
    d = d_ref[...]
    d = jnp.where(jnp.abs(d) < 1e3, d, 0.0).astype(jnp.bfloat16)
    h = jnp.dot(d, w1_ref[0].astype(jnp.bfloat16),
                preferred_element_type=jnp.float32)
    h = jax.nn.gelu(h + b1_ref[0])
    o_ref[...] = (jnp.dot(h.astype(jnp.bfloat16),
                          w2_ref[0].astype(jnp.bfloat16),
                          preferred_element_type=jnp.float32)
                  + b2_ref[0])


def _sc_gather(eo_hbm, gsl_hbm, out_hbm, idx_v, rows_v, sem):
    wid = lax.axis_index("s") * 2 + lax.axis_index("c")
    a_base = wid * 128
    pltpu.sync_copy(gsl_hbm.at[pl.ds(a_base, 128)], idx_v)
    pltpu.async_copy(eo_hbm.at[idx_v], rows_v, sem).wait()
    pltpu.sync_copy(rows_v, out_hbm.at[pl.ds(a_base, 128)])


def _k_wcombine(ga_ref, gb_ref, wa_ref, wb_ref, x2_ref, out_ref):
    out_ref[...] = (x2_ref[...] + wa_ref[...] * ga_ref[...]
                    + wb_ref[...] * gb_ref[...])


def kernel(x, g1, bn1, g2, bn2, Wq, Wk, Wv, Wo, Wg, W1, be1, W2, be2):
    f32 = jnp.float32
    g1r, bn1r = g1.reshape(1, -1), bn1.reshape(1, -1)
    g2r, bn2r = g2.reshape(1, -1), bn2.reshape(1, -1)
    wg_pad = jnp.pad(Wg, ((0, 0), (0, 128 - E)))
    be1r = be1.reshape(E, 1, D_FF)
    be2r = be2.reshape(E, 1, D_MODEL)

    full = lambda shp: pl.BlockSpec(shp, lambda *_: tuple(0 for _ in shp))

    q, k, v = pl.pallas_call(
        _k_ln_qkv,
        grid=(T // _RB,),
        in_specs=[
            pl.BlockSpec((_RB, D_MODEL), lambda i: (i, 0)),
            full((1, D_MODEL)), full((1, D_MODEL)),
            full((D_MODEL, D_MODEL)), full((D_MODEL, D_MODEL)),
            full((D_MODEL, D_MODEL)),
        ],
        out_specs=[pl.BlockSpec((_RB, D_MODEL), lambda i: (i, 0))] * 3,
        out_shape=[jax.ShapeDtypeStruct((T, D_MODEL), f32)] * 3,
    )(x, g1r, bn1r, Wq, Wk, Wv)

    o = pl.pallas_call(
        _k_attn,
        grid=(N_HEADS // 2,),
        in_specs=[pl.BlockSpec((T, 2 * HEAD_DIM), lambda h: (0, h))] * 3,
        out_specs=pl.BlockSpec((T, 2 * HEAD_DIM), lambda h: (0, h)),
        out_shape=jax.ShapeDtypeStruct((T, D_MODEL), f32),
    )(q, k, v)

    x2, mi, logits = pl.pallas_call(
        _k_proj_ln2_gate,
        grid=(T // _RB,),
        in_specs=[
            pl.BlockSpec((_RB, D_MODEL), lambda i: (i, 0)),
            full((D_MODEL, D_MODEL)),
            pl.BlockSpec((_RB, D_MODEL), lambda i: (i, 0)),
            full((1, D_MODEL)), full((1, D_MODEL)),
            full((D_MODEL, 128)),
        ],
        out_specs=[
            pl.BlockSpec((_RB, D_MODEL), lambda i: (i, 0)),
            pl.BlockSpec((_RB, D_MODEL), lambda i: (i, 0)),
            pl.BlockSpec((_RB, 128), lambda i: (i, 0)),
        ],
        out_shape=[
            jax.ShapeDtypeStruct((T, D_MODEL), f32),
            jax.ShapeDtypeStruct((T, D_MODEL), f32),
            jax.ShapeDtypeStruct((T, 128), f32),
        ],
    )(o, Wo, x, g2r, bn2r, wg_pad)

    slot_a, slot_b, w_a, w_b, counts, sg_a, sg_b = pl.pallas_call(
        _k_router,
        grid=(2,),
        in_specs=[pl.BlockSpec((TC, 128), lambda c: (c, 0))],
        out_specs=[pl.BlockSpec((TC, 1), lambda c: (c, 0))] * 4
        + [pl.BlockSpec((1, 1, E), lambda c: (c, 0, 0))]
        + [pl.BlockSpec((TC, 1), lambda c: (c, 0))] * 2,
        out_shape=[
            jax.ShapeDtypeStruct((T, 1), jnp.int32),
            jax.ShapeDtypeStruct((T, 1), jnp.int32),
            jax.ShapeDtypeStruct((T, 1), f32),
            jax.ShapeDtypeStruct((T, 1), f32),
            jax.ShapeDtypeStruct((2, 1, E), jnp.int32),
            jax.ShapeDtypeStruct((T, 1), jnp.int32),
            jax.ShapeDtypeStruct((T, 1), jnp.int32),
        ],
    )(logits)

    sa2 = slot_a.reshape(2, TC)
    sb2 = slot_b.reshape(2, TC)
    slots = jnp.stack([sa2, sb2], axis=1).reshape(4 * TC)
    disp = pl.kernel(
        _sc_dispatch,
        mesh=plsc.VectorSubcoreMesh(core_axis_name="c", subcore_axis_name="s"),
        out_type=jax.ShapeDtypeStruct((NDISP, D_MODEL), f32),
        scratch_types=[
            pltpu.VMEM((128,), jnp.int32),
            pltpu.VMEM((128, D_MODEL), f32),
            pltpu.SemaphoreType.DMA,
        ],
    )(mi, slots)

    eo = pl.pallas_call(
        _k_ffn,
        grid=(E, 2),
        in_specs=[
            pl.BlockSpec((CAP, D_MODEL), lambda e, c: (c * E + e, 0)),
            pl.BlockSpec((1, D_MODEL, D_FF), lambda e, c: (e, 0, 0)),
            pl.BlockSpec((1, 1, D_FF), lambda e, c: (e, 0, 0)),
            pl.BlockSpec((1, D_FF, D_MODEL), lambda e, c: (e, 0, 0)),
            pl.BlockSpec((1, 1, D_MODEL), lambda e, c: (e, 0, 0)),
        ],
        out_specs=pl.BlockSpec((CAP, D_MODEL), lambda e, c: (c * E + e, 0)),
        out_shape=jax.ShapeDtypeStruct((2 * NSLOT, D_MODEL), f32),
    )(disp, W1, be1r, W2, be2r)

    ga2 = sg_a.reshape(2, TC)
    gb2 = sg_b.reshape(2, TC)
    gslots = jnp.stack([ga2, gb2], axis=1).reshape(4 * TC)
    gab = pl.kernel(
        _sc_gather,
        mesh=plsc.VectorSubcoreMesh(core_axis_name="c", subcore_axis_name="s"),
        out_type=jax.ShapeDtypeStruct((4 * TC, D_MODEL), f32),
        scratch_types=[
            pltpu.VMEM((128,), jnp.int32),
            pltpu.VMEM((128, D_MODEL), f32),
            pltpu.SemaphoreType.DMA,
        ],
    )(eo, gslots)

    rb8 = 256
    nb8 = TC // rb8
    out = pl.pallas_call(
        _k_wcombine,
        grid=(2 * nb8,),
        in_specs=[
            pl.BlockSpec((rb8, D_MODEL), lambda i: ((i // nb8) * 2 * nb8 + i % nb8, 0)),
            pl.BlockSpec((rb8, D_MODEL), lambda i: ((i // nb8) * 2 * nb8 + nb8 + i % nb8, 0)),
            pl.BlockSpec((rb8, 1), lambda i: (i, 0)),
            pl.BlockSpec((rb8, 1), lambda i: (i, 0)),
            pl.BlockSpec((rb8, D_MODEL), lambda i: (i, 0)),
        ],
        out_specs=pl.BlockSpec((rb8, D_MODEL), lambda i: (i, 0)),
        out_shape=jax.ShapeDtypeStruct((T, D_MODEL), f32),
    )(gab, gab, w_a, w_b, x2)

    return out


# speedup vs baseline: 1.1423x; 1.0104x over previous
import functools

import jax
import jax.numpy as jnp
import numpy as np
from jax import lax
from jax.experimental import pallas as pl
from jax.experimental.pallas import tpu as pltpu
from jax.experimental.pallas import tpu_sc as plsc

D_MODEL = 768
N_HEADS = 12
HEAD_DIM = 64
E = 8
TOP_K = 2
D_FF = 3072
T = 2048
CAP = 512
TC = T // 2
NSLOT = E * CAP
SENT = 2 * NSLOT
NDISP = 2 * NSLOT + 512

_RB = 256


def _ln(x, g, b):
    m = jnp.mean(x, axis=-1, keepdims=True)
    v = jnp.mean((x - m) ** 2, axis=-1, keepdims=True)
    return (x - m) * jax.lax.rsqrt(v + 1e-5) * g + b


def _k_ln_qkv(x_ref, g_ref, b_ref, wq_ref, wk_ref, wv_ref,
              q_ref, k_ref, v_ref):
    h = _ln(x_ref[...], g_ref[...], b_ref[...])
    q_ref[...] = jnp.dot(h, wq_ref[...], preferred_element_type=jnp.float32)
    k_ref[...] = jnp.dot(h, wk_ref[...], preferred_element_type=jnp.float32)
    v_ref[...] = jnp.dot(h, wv_ref[...], preferred_element_type=jnp.float32)


def _k_attn(q_ref, k_ref, v_ref, o_ref):
    scale = (1.0 / np.sqrt(HEAD_DIM)).astype(np.float32)
    for p in range(2):
        sl = slice(p * HEAD_DIM, (p + 1) * HEAD_DIM)
        q = q_ref[:, sl] * scale
        k = k_ref[:, sl]
        v = v_ref[:, sl]
        s = jax.lax.dot_general(q, k, (((1,), (1,)), ((), ())),
                                preferred_element_type=jnp.float32)
        s = s - jnp.max(s, axis=-1, keepdims=True)
        e = jnp.exp(s)
        denom = jnp.sum(e, axis=-1, keepdims=True)
        o = jnp.dot(e, v, preferred_element_type=jnp.float32)
        o_ref[:, sl] = o * (1.0 / denom)


def _k_proj_ln2_router(o_ref, wo_ref, x_ref, g_ref, b_ref, wg_ref,
                       x2_ref, mi_ref, sa_ref, sb_ref, wa_ref, wb_ref,
                       sga_ref, sgb_ref):
    x2 = x_ref[...] + jnp.dot(o_ref[...], wo_ref[...],
                              preferred_element_type=jnp.float32)
    x2_ref[...] = x2
    mi = _ln(x2, g_ref[...], b_ref[...])
    mi_ref[...] = mi
    lg = jnp.dot(mi, wg_ref[...], preferred_element_type=jnp.float32)[:, :E]
    m = jnp.max(lg, axis=1, keepdims=True)
    ex = jnp.exp(lg - m)
    p = ex / jnp.sum(ex, axis=1, keepdims=True)

    ie = jax.lax.broadcasted_iota(jnp.int32, (TC, E), 1)
    w1 = jnp.max(p, axis=1, keepdims=True)
    a1 = jnp.min(jnp.where(p == w1, ie, E), axis=1, keepdims=True)
    p2 = jnp.where(ie == a1, -jnp.inf, p)
    w2 = jnp.max(p2, axis=1, keepdims=True)
    a2 = jnp.min(jnp.where(p2 == w2, ie, E), axis=1, keepdims=True)
    ws = w1 + w2

    oha = (ie == a1).astype(jnp.float32)
    ohb = (ie == a2).astype(jnp.float32)
    ir = jax.lax.broadcasted_iota(jnp.int32, (TC, TC), 0)
    ic = jax.lax.broadcasted_iota(jnp.int32, (TC, TC), 1)
    ltri = (ir > ic).astype(jnp.float32)
    cex = jnp.dot(ltri, oha + ohb, preferred_element_type=jnp.float32)
    pos_a = jnp.sum(cex * oha, axis=1, keepdims=True)
    pos_b = jnp.sum(cex * ohb, axis=1, keepdims=True)
    cofs = pl.program_id(0) * NSLOT
    keep_a = pos_a < CAP
    keep_b = pos_b < CAP
    slot_a = cofs + a1 * CAP + pos_a.astype(jnp.int32)
    slot_b = cofs + a2 * CAP + pos_b.astype(jnp.int32)
    sa_ref[...] = jnp.where(keep_a, slot_a, SENT)
    sb_ref[...] = jnp.where(keep_b, slot_b, SENT)
    sga_ref[...] = jnp.where(keep_a, slot_a, 0)
    sgb_ref[...] = jnp.where(keep_b, slot_b, 0)
    wa_ref[...] = jnp.where(keep_a, w1 / ws, 0.0)
    wb_ref[...] = jnp.where(keep_b, w2 / ws, 0.0)


def _sc_dispatch(mi_hbm, slots_hbm, disp_hbm, idx_v, rows_v, sem):
    wid = lax.axis_index("s") * 2 + lax.axis_index("c")
    a_base = wid * 128
    row_base = (a_base // (2 * TC)) * TC + a_base % TC
    pltpu.sync_copy(slots_hbm.at[pl.ds(a_base, 128)], idx_v)
    pltpu.sync_copy(mi_hbm.at[pl.ds(row_base, 128)], rows_v)
    pltpu.async_copy(rows_v, disp_hbm.at[idx_v], sem).wait()


def _k_ffn(d_ref, w1_ref, b1_ref, w2_ref, b2_ref, o_ref):
    d = d_ref[...]
    d = jnp.where(jnp.abs(d) < 1e3, d, 0.0).astype(jnp.bfloat16)
    h = jnp.dot(d, w1_ref[0].astype(jnp.bfloat16),
                preferred_element_type=jnp.float32)
    h = jax.nn.gelu(h + b1_ref[0])
    o_ref[...] = (jnp.dot(h.astype(jnp.bfloat16),
                          w2_ref[0].astype(jnp.bfloat16),
                          preferred_element_type=jnp.float32)
                  + b2_ref[0])


def _sc_gather(eo_hbm, gsl_hbm, out_hbm, idx_v, rows_v, sem):
    wid = lax.axis_index("s") * 2 + lax.axis_index("c")
    a_base = wid * 128
    pltpu.sync_copy(gsl_hbm.at[pl.ds(a_base, 128)], idx_v)
    pltpu.async_copy(eo_hbm.at[idx_v], rows_v, sem).wait()
    pltpu.sync_copy(rows_v, out_hbm.at[pl.ds(a_base, 128)])


def _k_wcombine(ga_ref, gb_ref, wa_ref, wb_ref, x2_ref, out_ref):
    out_ref[...] = (x2_ref[...] + wa_ref[...] * ga_ref[...]
                    + wb_ref[...] * gb_ref[...])


def kernel(x, g1, bn1, g2, bn2, Wq, Wk, Wv, Wo, Wg, W1, be1, W2, be2):
    f32 = jnp.float32
    g1r, bn1r = g1.reshape(1, -1), bn1.reshape(1, -1)
    g2r, bn2r = g2.reshape(1, -1), bn2.reshape(1, -1)
    wg_pad = jnp.pad(Wg, ((0, 0), (0, 128 - E)))
    be1r = be1.reshape(E, 1, D_FF)
    be2r = be2.reshape(E, 1, D_MODEL)

    full = lambda shp: pl.BlockSpec(shp, lambda *_: tuple(0 for _ in shp))

    q, k, v = pl.pallas_call(
        _k_ln_qkv,
        grid=(T // _RB,),
        in_specs=[
            pl.BlockSpec((_RB, D_MODEL), lambda i: (i, 0)),
            full((1, D_MODEL)), full((1, D_MODEL)),
            full((D_MODEL, D_MODEL)), full((D_MODEL, D_MODEL)),
            full((D_MODEL, D_MODEL)),
        ],
        out_specs=[pl.BlockSpec((_RB, D_MODEL), lambda i: (i, 0))] * 3,
        out_shape=[jax.ShapeDtypeStruct((T, D_MODEL), f32)] * 3,
    )(x, g1r, bn1r, Wq, Wk, Wv)

    o = pl.pallas_call(
        _k_attn,
        grid=(N_HEADS // 2,),
        in_specs=[pl.BlockSpec((T, 2 * HEAD_DIM), lambda h: (0, h))] * 3,
        out_specs=pl.BlockSpec((T, 2 * HEAD_DIM), lambda h: (0, h)),
        out_shape=jax.ShapeDtypeStruct((T, D_MODEL), f32),
    )(q, k, v)

    x2, mi, slot_a, slot_b, w_a, w_b, sg_a, sg_b = pl.pallas_call(
        _k_proj_ln2_router,
        grid=(2,),
        in_specs=[
            pl.BlockSpec((TC, D_MODEL), lambda c: (c, 0)),
            full((D_MODEL, D_MODEL)),
            pl.BlockSpec((TC, D_MODEL), lambda c: (c, 0)),
            full((1, D_MODEL)), full((1, D_MODEL)),
            full((D_MODEL, 128)),
        ],
        out_specs=[pl.BlockSpec((TC, D_MODEL), lambda c: (c, 0))] * 2
        + [pl.BlockSpec((TC, 1), lambda c: (c, 0))] * 6,
        out_shape=[
            jax.ShapeDtypeStruct((T, D_MODEL), f32),
            jax.ShapeDtypeStruct((T, D_MODEL), f32),
            jax.ShapeDtypeStruct((T, 1), jnp.int32),
            jax.ShapeDtypeStruct((T, 1), jnp.int32),
            jax.ShapeDtypeStruct((T, 1), f32),
            jax.ShapeDtypeStruct((T, 1), f32),
            jax.ShapeDtypeStruct((T, 1), jnp.int32),
            jax.ShapeDtypeStruct((T, 1), jnp.int32),
        ],
    )(o, Wo, x, g2r, bn2r, wg_pad)

    sa2 = slot_a.reshape(2, TC)
    sb2 = slot_b.reshape(2, TC)
    slots = jnp.stack([sa2, sb2], axis=1).reshape(4 * TC)
    disp = pl.kernel(
        _sc_dispatch,
        mesh=plsc.VectorSubcoreMesh(core_axis_name="c", subcore_axis_name="s"),
        out_type=jax.ShapeDtypeStruct((NDISP, D_MODEL), f32),
        scratch_types=[
            pltpu.VMEM((128,), jnp.int32),
            pltpu.VMEM((128, D_MODEL), f32),
            pltpu.SemaphoreType.DMA,
        ],
    )(mi, slots)

    eo = pl.pallas_call(
        _k_ffn,
        grid=(E, 2),
        in_specs=[
            pl.BlockSpec((CAP, D_MODEL), lambda e, c: (c * E + e, 0)),
            pl.BlockSpec((1, D_MODEL, D_FF), lambda e, c: (e, 0, 0)),
            pl.BlockSpec((1, 1, D_FF), lambda e, c: (e, 0, 0)),
            pl.BlockSpec((1, D_FF, D_MODEL), lambda e, c: (e, 0, 0)),
            pl.BlockSpec((1, 1, D_MODEL), lambda e, c: (e, 0, 0)),
        ],
        out_specs=pl.BlockSpec((CAP, D_MODEL), lambda e, c: (c * E + e, 0)),
        out_shape=jax.ShapeDtypeStruct((2 * NSLOT, D_MODEL), f32),
    )(disp, W1, be1r, W2, be2r)

    ga2 = sg_a.reshape(2, TC)
    gb2 = sg_b.reshape(2, TC)
    gslots = jnp.stack([ga2, gb2], axis=1).reshape(4 * TC)
    gab = pl.kernel(
        _sc_gather,
        mesh=plsc.VectorSubcoreMesh(core_axis_name="c", subcore_axis_name="s"),
        out_type=jax.ShapeDtypeStruct((4 * TC, D_MODEL), f32),
        scratch_types=[
            pltpu.VMEM((128,), jnp.int32),
            pltpu.VMEM((128, D_MODEL), f32),
            pltpu.SemaphoreType.DMA,
        ],
    )(eo, gslots)

    rb8 = 256
    nb8 = TC // rb8
    out = pl.pallas_call(
        _k_wcombine,
        grid=(2 * nb8,),
        in_specs=[
            pl.BlockSpec((rb8, D_MODEL), lambda i: ((i // nb8) * 2 * nb8 + i % nb8, 0)),
            pl.BlockSpec((rb8, D_MODEL), lambda i: ((i // nb8) * 2 * nb8 + nb8 + i % nb8, 0)),
            pl.BlockSpec((rb8, 1), lambda i: (i, 0)),
            pl.BlockSpec((rb8, 1), lambda i: (i, 0)),
            pl.BlockSpec((rb8, D_MODEL), lambda i: (i, 0)),
        ],
        out_specs=pl.BlockSpec((rb8, D_MODEL), lambda i: (i, 0)),
        out_shape=jax.ShapeDtypeStruct((T, D_MODEL), f32),
    )(gab, gab, w_a, w_b, x2)

    return out
